# baseline (speedup 1.0000x reference)
import functools

import jax
import jax.numpy as jnp
from jax import lax
from jax.experimental import pallas as pl
from jax.experimental.pallas import tpu as pltpu
from jax.experimental.pallas import tpu_sc as plsc

NC = 2
NS = 16
NW = NC * NS
L = 16

B = 16384
D = 64
W = 2 * D
BPW = B // NW
CHUNK = 128
NCHUNK = BPW // CHUNK
GPC = CHUNK // L


@functools.partial(
    pl.kernel,
    out_type=jax.ShapeDtypeStruct((B,), jnp.float32),
    mesh=plsc.VectorSubcoreMesh(core_axis_name="c", subcore_axis_name="s"),
    scratch_types=[
        pltpu.VMEM((NCHUNK, CHUNK), jnp.int32),
        pltpu.VMEM((NCHUNK, CHUNK), jnp.int32),
        pltpu.VMEM((BPW,), jnp.int32),
        pltpu.VMEM((BPW,), jnp.int32),
        pltpu.VMEM((2, CHUNK, W), jnp.float32),
        pltpu.VMEM((2, CHUNK, W), jnp.float32),
        pltpu.VMEM((BPW,), jnp.float32),
        pltpu.SemaphoreType.DMA,
        pltpu.SemaphoreType.DMA,
    ],
    compiler_params=pltpu.CompilerParams(needs_layout_passes=False),
)
def _dot_gather(user_hbm, item_hbm, uf_hbm, if_hbm, out_hbm,
                div_u, div_i, par_u, par_i, rows_u, rows_i, out_v,
                sem_u, sem_i):
    wid = lax.axis_index("s") * NC + lax.axis_index("c")
    base = wid * BPW

    pltpu.sync_copy(user_hbm.at[wid], div_u)
    pltpu.sync_copy(item_hbm.at[wid], div_i)
    for j in range(NCHUNK):
        for k in range(CHUNK // L):
            u = div_u[j, pl.ds(k * L, L)]
            i = div_i[j, pl.ds(k * L, L)]
            par_u[pl.ds(j * CHUNK + k * L, L)] = (u & 1) << 6
            par_i[pl.ds(j * CHUNK + k * L, L)] = (i & 1) << 6
            div_u[j, pl.ds(k * L, L)] = lax.shift_right_logical(u, 1)
            div_i[j, pl.ds(k * L, L)] = lax.shift_right_logical(i, 1)

    def fire(j):
        buf = j % 2
        cu = pltpu.async_copy(uf_hbm.at[div_u.at[j]], rows_u.at[buf], sem_u)
        ci = pltpu.async_copy(if_hbm.at[div_i.at[j]], rows_i.at[buf], sem_i)
        return cu, ci

    copies = {0: fire(0), 1: fire(1)}

    iota = lax.iota(jnp.int32, L)

    for j in range(NCHUNK):
        cu, ci = copies.pop(j)
        cu.wait()
        ci.wait()
        buf = j % 2

        def group_body(g, carry, j=j, buf=buf):
            rows = g * L + iota
            pvu = par_u[pl.ds(j * CHUNK + g * L, L)]
            pvi = par_i[pl.ds(j * CHUNK + g * L, L)]
            zero = jnp.zeros((L,), jnp.float32)

            def col_block(cb, accs):
                a0, a1 = accs
                bu = pvu + cb * L
                bi = pvi + cb * L
                for cc in range(L):
                    u16 = plsc.load_gather(rows_u.at[buf], [rows, bu + cc])
                    v16 = plsc.load_gather(rows_i.at[buf], [rows, bi + cc])
                    if cc % 2:
                        a1 = a1 + u16 * v16
                    else:
                        a0 = a0 + u16 * v16
                return a0, a1

            a0, a1 = lax.fori_loop(0, D // L, col_block, (zero, zero))
            out_v[pl.ds(j * CHUNK + g * L, L)] = a0 + a1
            return carry

        lax.fori_loop(0, GPC, group_body, 0)
        if j + 2 < NCHUNK:
            copies[j + 2] = fire(j + 2)

    pltpu.sync_copy(out_v, out_hbm.at[pl.ds(base, BPW)])


def kernel(user, item, user_factors, item_factors):
    user2 = user.astype(jnp.int32).reshape(NW, NCHUNK, CHUNK)
    item2 = item.astype(jnp.int32).reshape(NW, NCHUNK, CHUNK)
    uf2 = user_factors.reshape(-1, W)
    if2 = item_factors.reshape(-1, W)
    return _dot_gather(user2, item2, uf2, if2)


# speedup vs baseline: 1.0074x; 1.0074x over previous
import functools

import jax
import jax.numpy as jnp
from jax import lax
from jax.experimental import pallas as pl
from jax.experimental.pallas import tpu as pltpu
from jax.experimental.pallas import tpu_sc as plsc

NC = 2
NS = 16
NW = NC * NS
L = 16

B = 16384
D = 64
BPW = B // NW
CHUNK = 128
NCHUNK = BPW // CHUNK
GPC = CHUNK // L


@functools.partial(
    pl.kernel,
    out_type=jax.ShapeDtypeStruct((B,), jnp.float32),
    mesh=plsc.VectorSubcoreMesh(core_axis_name="c", subcore_axis_name="s"),
    scratch_types=[
        pltpu.VMEM((NCHUNK, CHUNK), jnp.int32),
        pltpu.VMEM((NCHUNK, CHUNK), jnp.int32),
        pltpu.VMEM((2, CHUNK, D), jnp.float32),
        pltpu.VMEM((2, CHUNK, D), jnp.float32),
        pltpu.VMEM((BPW,), jnp.float32),
        pltpu.SemaphoreType.DMA,
        pltpu.SemaphoreType.DMA,
    ],
    compiler_params=pltpu.CompilerParams(
        needs_layout_passes=False, use_tc_tiling_on_sc=False),
)
def _dot_gather(user_hbm, item_hbm, uf_hbm, if_hbm, out_hbm,
                idx_u, idx_i, rows_u, rows_i, out_v, sem_u, sem_i):
    wid = lax.axis_index("s") * NC + lax.axis_index("c")
    base = wid * BPW

    pltpu.sync_copy(user_hbm.at[wid], idx_u)
    pltpu.sync_copy(item_hbm.at[wid], idx_i)

    def fire(j):
        buf = j % 2
        cu = pltpu.async_copy(uf_hbm.at[idx_u.at[j]], rows_u.at[buf], sem_u)
        ci = pltpu.async_copy(if_hbm.at[idx_i.at[j]], rows_i.at[buf], sem_i)
        return cu, ci

    copies = {0: fire(0), 1: fire(1)}

    iota = lax.iota(jnp.int32, L)

    for j in range(NCHUNK):
        cu, ci = copies.pop(j)
        cu.wait()
        ci.wait()
        buf = j % 2

        def group_body(g, carry, j=j, buf=buf):
            rows = g * L + iota
            zero = jnp.zeros((L,), jnp.float32)

            def col_block(cb, accs):
                a0, a1 = accs
                cb16 = cb * L
                for cc in range(L):
                    cols = cb16 + cc + jnp.zeros((L,), jnp.int32)
                    u16 = plsc.load_gather(rows_u.at[buf], [rows, cols])
                    v16 = plsc.load_gather(rows_i.at[buf], [rows, cols])
                    if cc % 2:
                        a1 = a1 + u16 * v16
                    else:
                        a0 = a0 + u16 * v16
                return a0, a1

            a0, a1 = lax.fori_loop(0, D // L, col_block, (zero, zero))
            out_v[pl.ds(j * CHUNK + g * L, L)] = a0 + a1
            return carry

        lax.fori_loop(0, GPC, group_body, 0)
        if j + 2 < NCHUNK:
            copies[j + 2] = fire(j + 2)

    pltpu.sync_copy(out_v, out_hbm.at[pl.ds(base, BPW)])


def kernel(user, item, user_factors, item_factors):
    user2 = user.astype(jnp.int32).reshape(NW, NCHUNK, CHUNK)
    item2 = item.astype(jnp.int32).reshape(NW, NCHUNK, CHUNK)
    return _dot_gather(user2, item2, user_factors, item_factors)
